# baseline (speedup 1.0000x reference)
import functools

import jax
import jax.numpy as jnp
from jax.experimental import pallas as pl
from jax.experimental.pallas import tpu as pltpu
from jax.experimental.pallas import tpu_sc as plsc

VOCAB = 1000
EMB = 64
HID = 32
G4 = 4 * HID
NCLS = 10
B = 4096
L = 200

GATHER_WINDOW = 128
GATHER_DTYPE = jnp.float32
CHUNK_L = 10


def _project_kernel(emb_ref, wih_ref, p_ref):
    p_ref[...] = jnp.dot(
        emb_ref[...], wih_ref[...], preferred_element_type=jnp.float32
    ).astype(p_ref.dtype)


def _project_table(emb_table, wih_t, dtype):
    return pl.pallas_call(
        _project_kernel,
        out_shape=jax.ShapeDtypeStruct((VOCAB, G4), dtype),
    )(emb_table, wih_t)


def _sc_gather(table, idx_flat, out_d):
    n_idx = idx_flat.shape[0]
    d = table.shape[1]
    del out_d
    mesh = plsc.VectorSubcoreMesh(core_axis_name="c", subcore_axis_name="s")
    n_workers = 32
    per_w = n_idx // n_workers
    w = GATHER_WINDOW
    n_chunks = per_w // w

    @functools.partial(
        pl.kernel,
        out_type=jax.ShapeDtypeStruct((n_idx, d), table.dtype),
        mesh=mesh,
        scratch_types=[
            pltpu.VMEM((per_w,), jnp.int32),
            pltpu.VMEM((w, d), table.dtype),
            pltpu.VMEM((w, d), table.dtype),
            pltpu.SemaphoreType.DMA,
            pltpu.SemaphoreType.DMA,
        ],
    )
    def gather_kernel(table_hbm, idx_hbm, out_hbm, idx_v, rows0, rows1,
                      gsem0, gsem1):
        wid = jax.lax.axis_index("s") * 2 + jax.lax.axis_index("c")
        base = wid * per_w
        pltpu.sync_copy(idx_hbm.at[pl.ds(base, per_w)], idx_v)

        def start_gather(chunk, rows, sem):
            pltpu.async_copy(
                table_hbm.at[idx_v.at[pl.ds(chunk * w, w)]], rows, sem)

        def wait_gather(rows, sem):
            pltpu.make_async_copy(
                table_hbm.at[idx_v.at[pl.ds(0, w)]], rows, sem).wait()

        def writeout(rows, chunk):
            pltpu.sync_copy(rows, out_hbm.at[pl.ds(base + chunk * w, w)])

        start_gather(0, rows0, gsem0)

        @pl.loop(0, n_chunks, step=2)
        def _(ck):
            start_gather(ck + 1, rows1, gsem1)
            wait_gather(rows0, gsem0)
            writeout(rows0, ck)

            @pl.when(ck + 2 < n_chunks)
            def _():
                start_gather(ck + 2, rows0, gsem0)

            wait_gather(rows1, gsem1)
            writeout(rows1, ck + 1)

    return gather_kernel(table, idx_flat)


def _lstm_chunk_kernel(gx_ref, whh_ref, bias_ref, fcw_ref, fcb_ref, hin_ref,
                       cin_ref, hout_ref, cout_ref, out_ref):
    t = pl.program_id(0)

    @pl.when(t == 0)
    def _():
        hout_ref[...] = hin_ref[...]
        cout_ref[...] = cin_ref[...]

    h = hout_ref[...]
    gx_t = jnp.transpose(gx_ref[0].astype(jnp.float32))
    gates = gx_t + bias_ref[...] + jnp.dot(
        whh_ref[...], h, preferred_element_type=jnp.float32)

    def sigmoid(x):
        return 0.5 * jnp.tanh(0.5 * x) + 0.5

    i = sigmoid(gates[0 * HID:1 * HID, :])
    f = sigmoid(gates[1 * HID:2 * HID, :])
    g = jnp.tanh(gates[2 * HID:3 * HID, :])
    o = sigmoid(gates[3 * HID:4 * HID, :])
    c = f * cout_ref[...] + i * g
    h = o * jnp.tanh(c)
    cout_ref[...] = c
    hout_ref[...] = h

    @pl.when(t == CHUNK_L - 1)
    def _():
        out_t = jnp.dot(
            fcw_ref[...], h, preferred_element_type=jnp.float32)
        out_ref[...] = jnp.transpose(out_t) + fcb_ref[...]


def _tc_lstm_chunk(gx, whh, bias, fcw, fcb, h, c):
    return pl.pallas_call(
        _lstm_chunk_kernel,
        grid=(CHUNK_L,),
        in_specs=[
            pl.BlockSpec((1, B, G4), lambda t: (t, 0, 0)),
            pl.BlockSpec((G4, HID), lambda t: (0, 0)),
            pl.BlockSpec((G4, 1), lambda t: (0, 0)),
            pl.BlockSpec((NCLS, HID), lambda t: (0, 0)),
            pl.BlockSpec((1, NCLS), lambda t: (0, 0)),
            pl.BlockSpec((HID, B), lambda t: (0, 0)),
            pl.BlockSpec((HID, B), lambda t: (0, 0)),
        ],
        out_specs=[
            pl.BlockSpec((HID, B), lambda t: (0, 0)),
            pl.BlockSpec((HID, B), lambda t: (0, 0)),
            pl.BlockSpec((B, NCLS), lambda t: (0, 0)),
        ],
        out_shape=[
            jax.ShapeDtypeStruct((HID, B), jnp.float32),
            jax.ShapeDtypeStruct((HID, B), jnp.float32),
            jax.ShapeDtypeStruct((B, NCLS), jnp.float32),
        ],
        input_output_aliases={5: 0, 6: 1},
    )(gx, whh, bias, fcw, fcb, h, c)


@jax.jit
def kernel(text, emb_table, w_ih, w_hh, b_ih, b_hh, fc_w, fc_b):
    idx_flat = text.T.astype(jnp.int32).reshape(L * B)

    wih_t = w_ih.T
    proj = _project_table(emb_table, wih_t, jnp.float32)

    bias = (b_ih + b_hh).reshape(G4, 1)
    fcb = fc_b.reshape(1, NCLS)
    h = jnp.zeros((HID, B), jnp.float32)
    c = jnp.zeros((HID, B), jnp.float32)
    out = None
    for k in range(L // CHUNK_L):
        idx_k = jax.lax.dynamic_slice_in_dim(
            idx_flat, k * CHUNK_L * B, CHUNK_L * B)
        gx_k = _sc_gather(proj, idx_k, G4).reshape(CHUNK_L, B, G4)
        h, c, out = _tc_lstm_chunk(gx_k, w_hh, bias, fc_w, fcb, h, c)
    return out


# speedup vs baseline: 1.1931x; 1.1931x over previous
import functools

import jax
import jax.numpy as jnp
from jax.experimental import pallas as pl
from jax.experimental.pallas import tpu as pltpu
from jax.experimental.pallas import tpu_sc as plsc

VOCAB = 1000
EMB = 64
HID = 32
G4 = 4 * HID
NCLS = 10
B = 4096
L = 200

GATHER_WINDOW = 128
CHUNK_L = 20


def _project_kernel(emb_ref, wih_ref, p_ref):
    p_ref[...] = jnp.dot(
        emb_ref[...], wih_ref[...], preferred_element_type=jnp.float32
    ).astype(p_ref.dtype)


def _project_table(emb_table, wih_t, dtype):
    return pl.pallas_call(
        _project_kernel,
        out_shape=jax.ShapeDtypeStruct((VOCAB, G4), dtype),
    )(emb_table, wih_t)


def _sc_gather(table, idx_flat):
    n_idx = idx_flat.shape[0]
    d = table.shape[1]
    mesh = plsc.VectorSubcoreMesh(core_axis_name="c", subcore_axis_name="s")
    n_workers = 32
    per_w = n_idx // n_workers
    w = GATHER_WINDOW
    n_chunks = per_w // w

    @functools.partial(
        pl.kernel,
        out_type=jax.ShapeDtypeStruct((n_idx, d), table.dtype),
        mesh=mesh,
        scratch_types=[
            pltpu.VMEM((per_w,), jnp.int32),
            pltpu.VMEM((w, d), table.dtype),
            pltpu.VMEM((w, d), table.dtype),
            pltpu.SemaphoreType.DMA,
            pltpu.SemaphoreType.DMA,
        ],
    )
    def gather_kernel(table_hbm, idx_hbm, out_hbm, idx_v, rows0, rows1,
                      gsem0, gsem1):
        wid = jax.lax.axis_index("s") * 2 + jax.lax.axis_index("c")
        base = wid * per_w
        pltpu.sync_copy(idx_hbm.at[pl.ds(base, per_w)], idx_v)

        def start_gather(chunk, rows, sem):
            pltpu.async_copy(
                table_hbm.at[idx_v.at[pl.ds(chunk * w, w)]], rows, sem)

        def wait_gather(rows, sem):
            pltpu.make_async_copy(
                table_hbm.at[idx_v.at[pl.ds(0, w)]], rows, sem).wait()

        def writeout(rows, chunk):
            pltpu.sync_copy(rows, out_hbm.at[pl.ds(base + chunk * w, w)])

        start_gather(0, rows0, gsem0)

        @pl.loop(0, n_chunks, step=2)
        def _(ck):
            start_gather(ck + 1, rows1, gsem1)
            wait_gather(rows0, gsem0)
            writeout(rows0, ck)

            @pl.when(ck + 2 < n_chunks)
            def _():
                start_gather(ck + 2, rows0, gsem0)

            wait_gather(rows1, gsem1)
            writeout(rows1, ck + 1)

    return gather_kernel(table, idx_flat)


def _lstm_chunk_kernel(gx_ref, whh_ref, bias_ref, fcw_ref, fcb_ref, hin_ref,
                       cin_ref, hout_ref, cout_ref, out_ref):
    t = pl.program_id(0)

    @pl.when(t == 0)
    def _():
        hout_ref[...] = hin_ref[...]
        cout_ref[...] = cin_ref[...]

    h = hout_ref[...]
    gx_t = jnp.transpose(gx_ref[0].astype(jnp.float32))
    gates = gx_t + bias_ref[...] + jnp.dot(
        whh_ref[...], h, preferred_element_type=jnp.float32)

    def sigmoid(x):
        return 0.5 * jnp.tanh(0.5 * x) + 0.5

    i = sigmoid(gates[0 * HID:1 * HID, :])
    f = sigmoid(gates[1 * HID:2 * HID, :])
    g = jnp.tanh(gates[2 * HID:3 * HID, :])
    o = sigmoid(gates[3 * HID:4 * HID, :])
    c = f * cout_ref[...] + i * g
    h = o * jnp.tanh(c)
    cout_ref[...] = c
    hout_ref[...] = h

    @pl.when(t == CHUNK_L - 1)
    def _():
        out_t = jnp.dot(
            fcw_ref[...], h, preferred_element_type=jnp.float32)
        out_ref[...] = jnp.transpose(out_t) + fcb_ref[...]


def _tc_lstm_chunk(gx, whh, bias, fcw, fcb, h, c):
    return pl.pallas_call(
        _lstm_chunk_kernel,
        grid=(CHUNK_L,),
        in_specs=[
            pl.BlockSpec((1, B, G4), lambda t: (t, 0, 0)),
            pl.BlockSpec((G4, HID), lambda t: (0, 0)),
            pl.BlockSpec((G4, 1), lambda t: (0, 0)),
            pl.BlockSpec((NCLS, HID), lambda t: (0, 0)),
            pl.BlockSpec((1, NCLS), lambda t: (0, 0)),
            pl.BlockSpec((HID, B), lambda t: (0, 0)),
            pl.BlockSpec((HID, B), lambda t: (0, 0)),
        ],
        out_specs=[
            pl.BlockSpec((HID, B), lambda t: (0, 0)),
            pl.BlockSpec((HID, B), lambda t: (0, 0)),
            pl.BlockSpec((B, NCLS), lambda t: (0, 0)),
        ],
        out_shape=[
            jax.ShapeDtypeStruct((HID, B), jnp.float32),
            jax.ShapeDtypeStruct((HID, B), jnp.float32),
            jax.ShapeDtypeStruct((B, NCLS), jnp.float32),
        ],
        input_output_aliases={5: 0, 6: 1},
    )(gx, whh, bias, fcw, fcb, h, c)


@jax.jit
def kernel(text, emb_table, w_ih, w_hh, b_ih, b_hh, fc_w, fc_b):
    idx_flat = text.T.astype(jnp.int32).reshape(L * B)

    wih_t = w_ih.T
    proj = _project_table(emb_table, wih_t, jnp.float32)

    bias = (b_ih + b_hh).reshape(G4, 1)
    fcb = fc_b.reshape(1, NCLS)
    h = jnp.zeros((HID, B), jnp.float32)
    c = jnp.zeros((HID, B), jnp.float32)
    out = None
    for k in range(L // CHUNK_L):
        idx_k = jax.lax.dynamic_slice_in_dim(
            idx_flat, k * CHUNK_L * B, CHUNK_L * B)
        gx_k = _sc_gather(proj, idx_k).reshape(CHUNK_L, B, G4)
        h, c, out = _tc_lstm_chunk(gx_k, w_hh, bias, fc_w, fcb, h, c)
    return out
